# baseline (speedup 1.0000x reference)
import functools

import jax
import jax.numpy as jnp
from jax import lax
from jax.experimental import pallas as pl
from jax.experimental.pallas import tpu as pltpu
from jax.experimental.pallas import tpu_sc as plsc

D_MODEL = 1024
SEQ = 4096

_info = plsc.get_sparse_core_info()
_NC, _NS = _info.num_cores, _info.num_subcores
_NW = _NC * _NS
_ROWS_PER_W = SEQ // _NW

_mesh = plsc.VectorSubcoreMesh(core_axis_name="c", subcore_axis_name="s")


@functools.partial(
    pl.kernel,
    mesh=_mesh,
    out_type=jax.ShapeDtypeStruct((SEQ, D_MODEL), jnp.float32),
)
def _pe_slice_copy(pe_hbm, out_hbm):
    wid = lax.axis_index("s") * _NC + lax.axis_index("c")
    base = wid * _ROWS_PER_W
    pltpu.sync_copy(
        pe_hbm.at[pl.ds(base, _ROWS_PER_W)],
        out_hbm.at[pl.ds(base, _ROWS_PER_W)],
    )


def kernel(x, pe):
    del x
    return _pe_slice_copy(pe[0])[None]


# speedup vs baseline: 17.2500x; 17.2500x over previous
import functools

import jax
import jax.numpy as jnp
from jax import lax
from jax.experimental import pallas as pl
from jax.experimental.pallas import tpu as pltpu
from jax.experimental.pallas import tpu_sc as plsc

D_MODEL = 1024
SEQ = 4096

_info = plsc.get_sparse_core_info()
_NC, _NS = _info.num_cores, _info.num_subcores
_NW = _NC * _NS
_ROWS_PER_W = SEQ // _NW
_CHUNK = 64
_NCHUNK = _ROWS_PER_W // _CHUNK

_mesh = plsc.VectorSubcoreMesh(core_axis_name="c", subcore_axis_name="s")


@functools.partial(
    pl.kernel,
    mesh=_mesh,
    out_type=jax.ShapeDtypeStruct((SEQ, D_MODEL), jnp.float32),
    scratch_types=[
        pltpu.VMEM((_CHUNK, D_MODEL), jnp.float32),
        pltpu.SemaphoreType.DMA,
        pltpu.SemaphoreType.DMA,
    ],
)
def _pe_slice_copy(pe_hbm, out_hbm, buf, sem_in, sem_out):
    wid = lax.axis_index("s") * _NC + lax.axis_index("c")
    base = wid * _ROWS_PER_W
    out_handle = None
    for i in range(_NCHUNK):
        lo = base + i * _CHUNK
        if out_handle is not None:
            out_handle.wait()
        pltpu.async_copy(pe_hbm.at[pl.ds(lo, _CHUNK)], buf, sem_in).wait()
        out_handle = pltpu.async_copy(buf, out_hbm.at[pl.ds(lo, _CHUNK)], sem_out)
    out_handle.wait()


def kernel(x, pe):
    del x
    return _pe_slice_copy(pe[0])[None]
